# baseline (speedup 1.0000x reference)
import functools

import jax
import jax.numpy as jnp
from jax import lax
from jax.experimental import pallas as pl
from jax.experimental.pallas import tpu as pltpu
from jax.experimental.pallas import tpu_sc as plsc

NC = 2
NS = 16
NW = NC * NS
L = 16

ROWS, COLS = 32768, 1024
RV = COLS // L

R_SC = 4096
ROWS_PER_W = R_SC // NW
CHUNK_ROWS = 16
NCHUNK = ROWS_PER_W // CHUNK_ROWS
NBUF = 4
NGROUP = NCHUNK // NBUF
U = 8

BR = 512
TC_ROWS = ROWS - R_SC
TC_GRID = TC_ROWS // BR


def _chunk_max(buf, accs):
    @plsc.parallel_loop(0, CHUNK_ROWS, step=1, unroll=1, carry=tuple(accs))
    def folded(i, a):
        a = list(a)
        for j in range(RV):
            a[j % U] = jnp.maximum(a[j % U], buf[i, pl.ds(j * L, L)])
        return tuple(a)

    return list(folded)


_sc_mesh = plsc.VectorSubcoreMesh(core_axis_name="c", subcore_axis_name="s")


@functools.partial(
    pl.kernel,
    mesh=_sc_mesh,
    out_type=jax.ShapeDtypeStruct((NW, L), jnp.float32),
    scratch_types=[
        pltpu.VMEM((CHUNK_ROWS, COLS), jnp.float32),
        pltpu.VMEM((CHUNK_ROWS, COLS), jnp.float32),
        pltpu.VMEM((CHUNK_ROWS, COLS), jnp.float32),
        pltpu.VMEM((CHUNK_ROWS, COLS), jnp.float32),
        pltpu.VMEM((L,), jnp.float32),
        pltpu.SemaphoreType.DMA,
        pltpu.SemaphoreType.DMA,
        pltpu.SemaphoreType.DMA,
        pltpu.SemaphoreType.DMA,
    ],
)
def _sc_partial_max(vals, out_hbm, b0, b1, b2, b3, outv, s0, s1, s2, s3):
    wid = lax.axis_index("s") * NC + lax.axis_index("c")
    row0 = wid * ROWS_PER_W
    bufs = (b0, b1, b2, b3)
    sems = (s0, s1, s2, s3)

    def copy(g, b):
        return pltpu.make_async_copy(
            vals.at[pl.ds(row0 + g * CHUNK_ROWS, CHUNK_ROWS)], bufs[b], sems[b]
        )

    for b in range(NBUF):
        copy(b, b).start()

    neg_inf = jnp.full((L,), -jnp.inf, dtype=jnp.float32)

    def body(gg, accs):
        accs = list(accs)
        for b in range(NBUF):
            g = gg * NBUF + b
            copy(g, b).wait()
            accs = _chunk_max(bufs[b], accs)

            @pl.when(g + NBUF < NCHUNK)
            def _():
                copy(g + NBUF, b).start()

        return tuple(accs)

    accs = list(lax.fori_loop(0, NGROUP, body, (neg_inf,) * U))
    while len(accs) > 1:
        accs = [jnp.maximum(accs[i], accs[i + 1]) for i in range(0, len(accs), 2)]
    outv[...] = accs[0]
    pltpu.sync_copy(outv, out_hbm.at[wid])


U_TC = 4


def _tc_partial_kernel(x_ref, o_ref, acc):
    i = pl.program_id(0)

    @pl.when(i == 0)
    def _():
        acc[...] = jnp.full((U_TC * 8, COLS), -jnp.inf, dtype=jnp.float32)

    accs = [acc[pl.ds(u * 8, 8), :] for u in range(U_TC)]
    for k in range(BR // 8):
        accs[k % U_TC] = jnp.maximum(accs[k % U_TC], x_ref[pl.ds(k * 8, 8), :])
    for u in range(U_TC):
        acc[pl.ds(u * 8, 8), :] = accs[u]

    @pl.when(i == TC_GRID - 1)
    def _():
        a = accs
        while len(a) > 1:
            a = [jnp.maximum(a[j], a[j + 1]) for j in range(0, len(a), 2)]
        o_ref[...] = a[0]


def _tc_partial_max(values):
    return pl.pallas_call(
        _tc_partial_kernel,
        grid=(TC_GRID,),
        in_specs=[
            pl.BlockSpec((BR, COLS), lambda i: (i + R_SC // BR, 0))
        ],
        out_shape=jax.ShapeDtypeStruct((8, COLS), jnp.float32),
        out_specs=pl.BlockSpec((8, COLS), lambda i: (0, 0)),
        scratch_shapes=[pltpu.VMEM((U_TC * 8, COLS), jnp.float32)],
    )(values)


def _combine_kernel(parts_ref, tc_ref, o_ref):
    o_ref[0, 0] = jnp.maximum(jnp.max(parts_ref[...]), jnp.max(tc_ref[...]))


def kernel(values, prefix_sum):
    del prefix_sum
    sc_parts = _sc_partial_max(values)
    tc_part = _tc_partial_max(values)
    combined = pl.pallas_call(
        _combine_kernel,
        out_shape=jax.ShapeDtypeStruct((1, 1), jnp.float32),
        out_specs=pl.BlockSpec(memory_space=pltpu.SMEM),
    )(sc_parts, tc_part)
    return combined[0, 0]


# speedup vs baseline: 1.2288x; 1.2288x over previous
import functools

import jax
import jax.numpy as jnp
from jax import lax
from jax.experimental import pallas as pl
from jax.experimental.pallas import tpu as pltpu
from jax.experimental.pallas import tpu_sc as plsc

NC = 2
NS = 16
NW = NC * NS
L = 16

ROWS, COLS = 32768, 1024
RV = COLS // L

R_SC = 4096
ROWS_PER_W = R_SC // NW
CHUNK_ROWS = 16
NCHUNK = ROWS_PER_W // CHUNK_ROWS
NBUF = 4
NGROUP = NCHUNK // NBUF
U = 8

BR = 512
TC_ROWS = ROWS - R_SC
TC_GRID = TC_ROWS // BR


def _chunk_max(buf, accs):
    @plsc.parallel_loop(0, CHUNK_ROWS, step=1, unroll=4, carry=tuple(accs))
    def folded(i, a):
        a = list(a)
        for j in range(RV):
            a[j % U] = jnp.maximum(a[j % U], buf[i, pl.ds(j * L, L)])
        return tuple(a)

    return list(folded)


_sc_mesh = plsc.VectorSubcoreMesh(core_axis_name="c", subcore_axis_name="s")


@functools.partial(
    pl.kernel,
    mesh=_sc_mesh,
    out_type=jax.ShapeDtypeStruct((NW, L), jnp.float32),
    scratch_types=[
        pltpu.VMEM((CHUNK_ROWS, COLS), jnp.float32),
        pltpu.VMEM((CHUNK_ROWS, COLS), jnp.float32),
        pltpu.VMEM((CHUNK_ROWS, COLS), jnp.float32),
        pltpu.VMEM((CHUNK_ROWS, COLS), jnp.float32),
        pltpu.VMEM((L,), jnp.float32),
        pltpu.SemaphoreType.DMA,
        pltpu.SemaphoreType.DMA,
        pltpu.SemaphoreType.DMA,
        pltpu.SemaphoreType.DMA,
    ],
)
def _sc_partial_max(vals, out_hbm, b0, b1, b2, b3, outv, s0, s1, s2, s3):
    wid = lax.axis_index("s") * NC + lax.axis_index("c")
    row0 = wid * ROWS_PER_W
    bufs = (b0, b1, b2, b3)
    sems = (s0, s1, s2, s3)

    def copy(g, b):
        return pltpu.make_async_copy(
            vals.at[pl.ds(row0 + g * CHUNK_ROWS, CHUNK_ROWS)], bufs[b], sems[b]
        )

    for b in range(NBUF):
        copy(b, b).start()

    neg_inf = jnp.full((L,), -jnp.inf, dtype=jnp.float32)

    def body(gg, accs):
        accs = list(accs)
        for b in range(NBUF):
            g = gg * NBUF + b
            copy(g, b).wait()
            accs = _chunk_max(bufs[b], accs)

            @pl.when(g + NBUF < NCHUNK)
            def _():
                copy(g + NBUF, b).start()

        return tuple(accs)

    accs = list(lax.fori_loop(0, NGROUP, body, (neg_inf,) * U))
    while len(accs) > 1:
        accs = [jnp.maximum(accs[i], accs[i + 1]) for i in range(0, len(accs), 2)]
    outv[...] = accs[0]
    pltpu.sync_copy(outv, out_hbm.at[wid])


NBUF_TC = 4
U_TC = 4


def _tc_fold_chunk(buf, accs, br):
    accs = list(accs)
    for i in range(br // 8):
        accs[i % U_TC] = jnp.maximum(accs[i % U_TC], buf[pl.ds(i * 8, 8), :])
    return tuple(accs)


def _make_tc_kernel(row0, nrows, br):
    nchunk = nrows // br
    ngroup = nchunk // NBUF_TC

    def tc_kernel(x_hbm, o_ref, *refs):
        bufs = refs[:NBUF_TC]
        sems = refs[NBUF_TC:]

        def copy(g, b):
            return pltpu.make_async_copy(
                x_hbm.at[pl.ds(row0 + g * br, br)], bufs[b], sems[b]
            )

        for b in range(NBUF_TC):
            copy(b, b).start()

        def body(gg, accs):
            for b in range(NBUF_TC):
                g = gg * NBUF_TC + b
                copy(g, b).wait()
                accs = _tc_fold_chunk(bufs[b], accs, br)

                @pl.when(g + NBUF_TC < nchunk)
                def _():
                    copy(g + NBUF_TC, b).start()

            return accs

        init = jnp.full((8, COLS), -jnp.inf, dtype=jnp.float32)
        accs = list(lax.fori_loop(0, ngroup, body, (init,) * U_TC))
        while len(accs) > 1:
            accs = [
                jnp.maximum(accs[i], accs[i + 1]) for i in range(0, len(accs), 2)
            ]
        o_ref[...] = accs[0]

    return tc_kernel


def _tc_partial_max(values, row0, nrows, br):
    return pl.pallas_call(
        _make_tc_kernel(row0, nrows, br),
        in_specs=[pl.BlockSpec(memory_space=pl.ANY)],
        out_shape=jax.ShapeDtypeStruct((8, COLS), jnp.float32),
        scratch_shapes=[pltpu.VMEM((br, COLS), jnp.float32)] * NBUF_TC
        + [pltpu.SemaphoreType.DMA] * NBUF_TC,
    )(values)


def _combine_kernel(parts_ref, tc_ref, o_ref):
    o_ref[0, 0] = jnp.maximum(jnp.max(parts_ref[...]), jnp.max(tc_ref[...]))


def kernel(values, prefix_sum):
    del prefix_sum
    sc_parts = _sc_partial_max(values)
    tc_part = _tc_partial_max(values, R_SC, TC_ROWS, BR)
    combined = pl.pallas_call(
        _combine_kernel,
        out_shape=jax.ShapeDtypeStruct((1, 1), jnp.float32),
        out_specs=pl.BlockSpec(memory_space=pltpu.SMEM),
    )(sc_parts, tc_part)
    return combined[0, 0]


# speedup vs baseline: 1.2340x; 1.0042x over previous
import functools

import jax
import jax.numpy as jnp
from jax import lax
from jax.experimental import pallas as pl
from jax.experimental.pallas import tpu as pltpu
from jax.experimental.pallas import tpu_sc as plsc

NC = 2
NS = 16
NW = NC * NS
L = 16

ROWS, COLS = 32768, 1024
RV = COLS // L

R_SC = 6144
ROWS_PER_W = R_SC // NW
CHUNK_ROWS = 32
NCHUNK = ROWS_PER_W // CHUNK_ROWS
NBUF = 2
NGROUP = NCHUNK // NBUF
U = 8

BR = 512
TC_ROWS = ROWS - R_SC
TC_GRID = TC_ROWS // BR


def _chunk_max(buf, accs):
    @plsc.parallel_loop(0, CHUNK_ROWS, step=1, unroll=4, carry=tuple(accs))
    def folded(i, a):
        a = list(a)
        for j in range(RV):
            a[j % U] = jnp.maximum(a[j % U], buf[i, pl.ds(j * L, L)])
        return tuple(a)

    return list(folded)


_sc_mesh = plsc.VectorSubcoreMesh(core_axis_name="c", subcore_axis_name="s")


@functools.partial(
    pl.kernel,
    mesh=_sc_mesh,
    out_type=jax.ShapeDtypeStruct((NW, L), jnp.float32),
    scratch_types=[pltpu.VMEM((CHUNK_ROWS, COLS), jnp.float32)] * NBUF
    + [pltpu.VMEM((L,), jnp.float32)]
    + [pltpu.SemaphoreType.DMA] * NBUF,
)
def _sc_partial_max(vals, out_hbm, *refs):
    bufs = refs[:NBUF]
    outv = refs[NBUF]
    sems = refs[NBUF + 1 :]
    wid = lax.axis_index("s") * NC + lax.axis_index("c")
    row0 = wid * ROWS_PER_W

    def copy(g, b):
        return pltpu.make_async_copy(
            vals.at[pl.ds(row0 + g * CHUNK_ROWS, CHUNK_ROWS)], bufs[b], sems[b]
        )

    for b in range(NBUF):
        copy(b, b).start()

    neg_inf = jnp.full((L,), -jnp.inf, dtype=jnp.float32)

    def body(gg, accs):
        accs = list(accs)
        for b in range(NBUF):
            g = gg * NBUF + b
            copy(g, b).wait()
            accs = _chunk_max(bufs[b], accs)

            @pl.when(g + NBUF < NCHUNK)
            def _():
                copy(g + NBUF, b).start()

        return tuple(accs)

    accs = list(lax.fori_loop(0, NGROUP, body, (neg_inf,) * U))
    while len(accs) > 1:
        accs = [jnp.maximum(accs[i], accs[i + 1]) for i in range(0, len(accs), 2)]
    outv[...] = accs[0]
    pltpu.sync_copy(outv, out_hbm.at[wid])


NBUF_TC = 4
U_TC = 4


def _tc_fold_chunk(buf, accs, br):
    accs = list(accs)
    for i in range(br // 8):
        accs[i % U_TC] = jnp.maximum(accs[i % U_TC], buf[pl.ds(i * 8, 8), :])
    return tuple(accs)


def _make_tc_kernel(row0, nrows, br):
    nchunk = nrows // br
    ngroup = nchunk // NBUF_TC

    def tc_kernel(x_hbm, o_ref, *refs):
        bufs = refs[:NBUF_TC]
        sems = refs[NBUF_TC:]

        def copy(g, b):
            return pltpu.make_async_copy(
                x_hbm.at[pl.ds(row0 + g * br, br)], bufs[b], sems[b]
            )

        for b in range(NBUF_TC):
            copy(b, b).start()

        def body(gg, accs):
            for b in range(NBUF_TC):
                g = gg * NBUF_TC + b
                copy(g, b).wait()
                accs = _tc_fold_chunk(bufs[b], accs, br)

                @pl.when(g + NBUF_TC < nchunk)
                def _():
                    copy(g + NBUF_TC, b).start()

            return accs

        init = jnp.full((8, COLS), -jnp.inf, dtype=jnp.float32)
        accs = list(lax.fori_loop(0, ngroup, body, (init,) * U_TC))
        while len(accs) > 1:
            accs = [
                jnp.maximum(accs[i], accs[i + 1]) for i in range(0, len(accs), 2)
            ]
        o_ref[...] = accs[0]

    return tc_kernel


def _tc_partial_max(values, row0, nrows, br):
    return pl.pallas_call(
        _make_tc_kernel(row0, nrows, br),
        in_specs=[pl.BlockSpec(memory_space=pl.ANY)],
        out_shape=jax.ShapeDtypeStruct((8, COLS), jnp.float32),
        scratch_shapes=[pltpu.VMEM((br, COLS), jnp.float32)] * NBUF_TC
        + [pltpu.SemaphoreType.DMA] * NBUF_TC,
    )(values)


def _combine_kernel(parts_ref, tc_ref, o_ref):
    o_ref[0, 0] = jnp.maximum(jnp.max(parts_ref[...]), jnp.max(tc_ref[...]))


def kernel(values, prefix_sum):
    del prefix_sum
    sc_parts = _sc_partial_max(values)
    tc_part = _tc_partial_max(values, R_SC, TC_ROWS, BR)
    combined = pl.pallas_call(
        _combine_kernel,
        out_shape=jax.ShapeDtypeStruct((1, 1), jnp.float32),
        out_specs=pl.BlockSpec(memory_space=pltpu.SMEM),
    )(sc_parts, tc_part)
    return combined[0, 0]


# speedup vs baseline: 1.7620x; 1.4279x over previous
import functools

import jax
import jax.numpy as jnp
from jax import lax
from jax.experimental import pallas as pl
from jax.experimental.pallas import tpu as pltpu
from jax.experimental.pallas import tpu_sc as plsc

NC = 2
NS = 16
NW = NC * NS
L = 16

ROWS, COLS = 32768, 1024
RV = COLS // L

R_SC = 6144
ROWS_PER_W = R_SC // NW
CHUNK_ROWS = 32
NCHUNK = ROWS_PER_W // CHUNK_ROWS
NBUF = 2
NGROUP = NCHUNK // NBUF
U = 8

BR = 512
TC_ROWS = ROWS - R_SC
TC_GRID = TC_ROWS // BR


def _chunk_max(buf, accs):
    @plsc.parallel_loop(0, CHUNK_ROWS, step=1, unroll=4, carry=tuple(accs))
    def folded(i, a):
        a = list(a)
        for j in range(RV):
            a[j % U] = jnp.maximum(a[j % U], buf[i, pl.ds(j * L, L)])
        return tuple(a)

    return list(folded)


_sc_mesh = plsc.VectorSubcoreMesh(core_axis_name="c", subcore_axis_name="s")


@functools.partial(
    pl.kernel,
    mesh=_sc_mesh,
    out_type=jax.ShapeDtypeStruct((NW, L), jnp.float32),
    scratch_types=[pltpu.VMEM((CHUNK_ROWS, COLS), jnp.float32)] * NBUF
    + [pltpu.VMEM((L,), jnp.float32)]
    + [pltpu.SemaphoreType.DMA] * NBUF,
)
def _sc_partial_max(vals, out_hbm, *refs):
    bufs = refs[:NBUF]
    outv = refs[NBUF]
    sems = refs[NBUF + 1 :]
    wid = lax.axis_index("s") * NC + lax.axis_index("c")
    row0 = wid * ROWS_PER_W

    def copy(g, b):
        return pltpu.make_async_copy(
            vals.at[pl.ds(row0 + g * CHUNK_ROWS, CHUNK_ROWS)], bufs[b], sems[b]
        )

    for b in range(NBUF):
        copy(b, b).start()

    neg_inf = jnp.full((L,), -jnp.inf, dtype=jnp.float32)

    def body(gg, accs):
        accs = list(accs)
        for b in range(NBUF):
            g = gg * NBUF + b
            copy(g, b).wait()
            accs = _chunk_max(bufs[b], accs)

            @pl.when(g + NBUF < NCHUNK)
            def _():
                copy(g + NBUF, b).start()

        return tuple(accs)

    accs = list(lax.fori_loop(0, NGROUP, body, (neg_inf,) * U))
    while len(accs) > 1:
        accs = [jnp.maximum(accs[i], accs[i + 1]) for i in range(0, len(accs), 2)]
    outv[...] = accs[0]
    pltpu.sync_copy(outv, out_hbm.at[wid])


NBUF_TC = 4
U_TC = 4


def _tc_fold_chunk(buf, accs, br):
    accs = list(accs)
    for i in range(br // 8):
        accs[i % U_TC] = jnp.maximum(accs[i % U_TC], buf[pl.ds(i * 8, 8), :])
    return tuple(accs)


def _make_tc_kernel(row0, nrows, br):
    nchunk = nrows // br
    ngroup = nchunk // NBUF_TC

    def tc_kernel(x_hbm, o_ref, *refs):
        bufs = refs[:NBUF_TC]
        sems = refs[NBUF_TC:]

        def copy(g, b):
            return pltpu.make_async_copy(
                x_hbm.at[pl.ds(row0 + g * br, br)], bufs[b], sems[b]
            )

        for b in range(NBUF_TC):
            copy(b, b).start()

        def body(gg, accs):
            for b in range(NBUF_TC):
                g = gg * NBUF_TC + b
                copy(g, b).wait()
                accs = _tc_fold_chunk(bufs[b], accs, br)

                @pl.when(g + NBUF_TC < nchunk)
                def _():
                    copy(g + NBUF_TC, b).start()

            return accs

        init = jnp.full((8, COLS), -jnp.inf, dtype=jnp.float32)
        accs = list(lax.fori_loop(0, ngroup, body, (init,) * U_TC))
        while len(accs) > 1:
            accs = [
                jnp.maximum(accs[i], accs[i + 1]) for i in range(0, len(accs), 2)
            ]
        o_ref[0, 0] = jnp.max(accs[0])

    return tc_kernel


def _tc_partial_max(values, row0, nrows, br):
    return pl.pallas_call(
        _make_tc_kernel(row0, nrows, br),
        in_specs=[pl.BlockSpec(memory_space=pl.ANY)],
        out_shape=jax.ShapeDtypeStruct((1, 1), jnp.float32),
        out_specs=pl.BlockSpec(memory_space=pltpu.SMEM),
        scratch_shapes=[pltpu.VMEM((br, COLS), jnp.float32)] * NBUF_TC
        + [pltpu.SemaphoreType.DMA] * NBUF_TC,
    )(values)


def _combine_kernel(parts_ref, tc_ref, o_ref):
    o_ref[0, 0] = jnp.maximum(jnp.max(parts_ref[...]), jnp.max(tc_ref[...]))


def kernel(values, prefix_sum):
    del prefix_sum
    tc_part = _tc_partial_max(values, 0, ROWS, BR)
    return tc_part[0, 0]
